# baseline (speedup 1.0000x reference)
import functools

import jax
import jax.numpy as jnp
from jax.experimental import pallas as pl
from jax.experimental.pallas import tpu as pltpu

NUM_ROWS = 48
WIDTH = 4096


def _dma_body(idx_ref, table_ref, out_ref, sem):
    row = idx_ref[0]
    pltpu.make_async_copy(table_ref.at[pl.ds(row, 1)], out_ref, sem).start()
    pltpu.make_async_copy(table_ref.at[pl.ds(row, 1)], out_ref, sem).wait()


def kernel(layer, layer_embedding):
    idx = jnp.asarray(layer, jnp.int32).reshape(1)
    table = layer_embedding.reshape(NUM_ROWS, WIDTH)
    out = pl.pallas_call(
        _dma_body,
        grid_spec=pltpu.PrefetchScalarGridSpec(
            num_scalar_prefetch=1,
            grid=(),
            in_specs=[pl.BlockSpec(memory_space=pl.ANY)],
            out_specs=pl.BlockSpec(memory_space=pl.ANY),
            scratch_shapes=[pltpu.SemaphoreType.DMA],
        ),
        out_shape=jax.ShapeDtypeStruct((1, WIDTH), jnp.float32),
    )(idx, table)
    return out.reshape(1, 1, WIDTH)
